# baseline (speedup 1.0000x reference)
import functools

import jax
import jax.numpy as jnp
from jax import lax
from jax.experimental import pallas as pl
from jax.experimental.pallas import tpu as pltpu
from jax.experimental.pallas import tpu_sc as plsc

NC = 2
NS = 16
NW = NC * NS
CH = 128
NBUF = 2
TB = 1280


def _mesh():
    return plsc.VectorSubcoreMesh(
        core_axis_name="c", subcore_axis_name="s", num_cores=NC,
        num_subcores=NS)


@functools.lru_cache(maxsize=None)
def _sc_degree(n_pad, nch):
    rows_pt = n_pad // NS

    def body(row_hbm, ones_hbm, zrow_hbm, degp_hbm, ones_v, idx_v, deg_sh,
             sem):
        c = lax.axis_index("c")
        s = lax.axis_index("s")
        wid = s * NC + c
        pltpu.sync_copy(zrow_hbm, deg_sh.at[pl.ds(s * rows_pt, rows_pt)])
        pltpu.sync_copy(ones_hbm, ones_v)
        pltpu.sync_copy(row_hbm.at[wid], idx_v)
        plsc.subcore_barrier()

        def group(g, carry):
            for k in range(8):
                pltpu.async_copy(ones_v, deg_sh.at[idx_v.at[g * 8 + k]], sem,
                                 add=True)
            for k in range(8):
                pltpu.make_async_copy(
                    ones_v, deg_sh.at[idx_v.at[g * 8 + k]], sem).wait()
            return carry

        lax.fori_loop(0, nch // 8, group, 0)
        plsc.subcore_barrier()
        pltpu.sync_copy(deg_sh.at[pl.ds(s * rows_pt, rows_pt)],
                        degp_hbm.at[pl.ds(c * n_pad + s * rows_pt, rows_pt)])

    return pl.kernel(
        body,
        out_type=jax.ShapeDtypeStruct((NC * n_pad,), jnp.float32),
        mesh=_mesh(),
        scratch_types=[
            pltpu.VMEM((CH,), jnp.float32),
            pltpu.VMEM((nch, CH), jnp.int32),
            pltpu.VMEM_SHARED((n_pad,), jnp.float32),
            pltpu.SemaphoreType.DMA,
        ],
    )


@functools.lru_cache(maxsize=None)
def _sc_spmm(n_pad, f, nch):
    rows_pt = n_pad // NS
    zch = rows_pt // 5

    def body(tab_hbm, row_hbm, col_hbm, zblk_hbm, outp_hbm, cidx_v, ridx_v,
             bufs, acc_sh, csem, gsems, rsems, ssems):
        c = lax.axis_index("c")
        s = lax.axis_index("s")
        wid = s * NC + c
        cpre = pltpu.async_copy(col_hbm.at[wid], cidx_v, csem)
        for k in range(NBUF):
            pltpu.async_copy(row_hbm.at[wid, k], ridx_v.at[k], rsems[k])
        for j in range(5):
            pltpu.sync_copy(zblk_hbm,
                            acc_sh.at[pl.ds(s * rows_pt + j * zch, zch)])
        cpre.wait()
        plsc.subcore_barrier()

        for k in range(NBUF):
            pltpu.async_copy(tab_hbm.at[cidx_v.at[k]], bufs[k], gsems[k])

        def pair(j, carry):
            base = j * NBUF
            for k in range(NBUF):
                ch = base + k
                pltpu.make_async_copy(
                    tab_hbm.at[cidx_v.at[ch]], bufs[k], gsems[k]).wait()
                pltpu.make_async_copy(
                    row_hbm.at[wid, ch], ridx_v.at[k], rsems[k]).wait()
                pltpu.async_copy(bufs[k], acc_sh.at[ridx_v.at[k]], ssems[k],
                                 add=True)
            for k in range(NBUF):
                ch = base + k
                pltpu.make_async_copy(bufs[k], acc_sh.at[ridx_v.at[k]],
                                      ssems[k]).wait()

                @pl.when(ch + NBUF < nch)
                def _():
                    pltpu.async_copy(tab_hbm.at[cidx_v.at[ch + NBUF]],
                                     bufs[k], gsems[k])
                    pltpu.async_copy(row_hbm.at[wid, ch + NBUF],
                                     ridx_v.at[k], rsems[k])
            return carry

        lax.fori_loop(0, nch // NBUF, pair, 0)
        plsc.subcore_barrier()
        for j in range(5):
            src_off = s * rows_pt + j * zch
            pltpu.sync_copy(
                acc_sh.at[pl.ds(src_off, zch)],
                outp_hbm.at[pl.ds(c * n_pad + src_off, zch)])

    return pl.kernel(
        body,
        out_type=jax.ShapeDtypeStruct((NC * n_pad, f), jnp.float32),
        mesh=_mesh(),
        scratch_types=[
            pltpu.VMEM((nch, CH), jnp.int32),
            pltpu.VMEM((NBUF, CH), jnp.int32),
            [pltpu.VMEM((CH, f), jnp.float32)] * NBUF,
            pltpu.VMEM_SHARED((n_pad, f), jnp.float32),
            pltpu.SemaphoreType.DMA,
            [pltpu.SemaphoreType.DMA] * NBUF,
            [pltpu.SemaphoreType.DMA] * NBUF,
            [pltpu.SemaphoreType.DMA] * NBUF,
        ],
    )


def _dis(degpt_ref):
    d = degpt_ref[:, 0:1] + degpt_ref[:, 1:2]
    return jnp.where(d > 0, lax.rsqrt(d), 0.0)


_HI = jax.lax.Precision.HIGHEST


def _prep_body(x_ref, degpt_ref, u_ref):
    u_ref[...] = x_ref[...] * _dis(degpt_ref)


def _mid_body(vp0_ref, vp1_ref, degpt_ref, t1_ref, w_ref):
    dis = _dis(degpt_ref)
    t1 = -((vp0_ref[...] + vp1_ref[...]) * dis)
    t1_ref[...] = t1
    w_ref[...] = t1 * dis


def _final_body(x_ref, t1_ref, zp0_ref, zp1_ref, degpt_ref, wt_ref, b_ref,
                o_ref):
    dis = _dis(degpt_ref)
    srow = -((zp0_ref[...] + zp1_ref[...]) * dis)
    acc = jnp.dot(x_ref[...], wt_ref[0] - wt_ref[2],
                  preferred_element_type=jnp.float32, precision=_HI)
    acc = acc + jnp.dot(t1_ref[...], wt_ref[1],
                        preferred_element_type=jnp.float32, precision=_HI)
    acc = acc + jnp.dot(srow, 2.0 * wt_ref[2],
                        preferred_element_type=jnp.float32, precision=_HI)
    o_ref[...] = acc + b_ref[...]


def _row_blk(f):
    return pl.BlockSpec((TB, f), lambda i: (i, 0))


def _row_blk_hi(n_pad, f):
    return pl.BlockSpec((TB, f), lambda i: (i + n_pad // TB, 0))


def kernel(x, index, weight, bias):
    n, f = x.shape
    e = index.shape[1]
    blk = NS * 8 * 5 * 2
    n_pad = ((n + blk - 1) // blk) * blk
    nch = -(-e // (NW * CH))
    nch = ((nch + 7) // 8) * 8
    e_pad = NW * nch * CH
    pad_i = jnp.arange(e_pad - e, dtype=jnp.int32)
    row = jnp.concatenate(
        [index[0], n + pad_i % (n_pad - n)]).reshape(NW, nch, CH)
    col = jnp.concatenate([index[1], pad_i % n]).reshape(NW, nch, CH)
    xp = jnp.pad(x, ((0, n_pad - n), (0, 0)))
    grid = (n_pad // TB,)

    ones_row = jnp.ones((CH,), jnp.float32)
    zero_row = jnp.zeros((n_pad // NS,), jnp.float32)
    zero_blk = jnp.zeros((n_pad // NS // 5, f), jnp.float32)

    degp = _sc_degree(n_pad, nch)(row, ones_row, zero_row)
    degpt = jnp.stack([degp[:n_pad], degp[n_pad:]], axis=1)

    wt_spec = pl.BlockSpec((3, f, f), lambda i: (0, 0, 0))

    u = pl.pallas_call(
        _prep_body,
        grid=grid,
        in_specs=[_row_blk(f), _row_blk(2)],
        out_specs=_row_blk(f),
        out_shape=jax.ShapeDtypeStruct((n_pad, f), jnp.float32),
    )(xp, degpt)

    vp = _sc_spmm(n_pad, f, nch)(u, row, col, zero_blk)

    t1, w = pl.pallas_call(
        _mid_body,
        grid=grid,
        in_specs=[_row_blk(f), _row_blk_hi(n_pad, f), _row_blk(2)],
        out_specs=(_row_blk(f), _row_blk(f)),
        out_shape=(jax.ShapeDtypeStruct((n_pad, f), jnp.float32),
                   jax.ShapeDtypeStruct((n_pad, f), jnp.float32)),
    )(vp, vp, degpt)

    zp = _sc_spmm(n_pad, f, nch)(w, row, col, zero_blk)

    out = pl.pallas_call(
        _final_body,
        grid=grid,
        in_specs=[_row_blk(f), _row_blk(f), _row_blk(f),
                  _row_blk_hi(n_pad, f), _row_blk(2), wt_spec,
                  pl.BlockSpec((1, f), lambda i: (0, 0))],
        out_specs=_row_blk(f),
        out_shape=jax.ShapeDtypeStruct((n_pad, f), jnp.float32),
    )(xp, t1, zp, zp, degpt, weight, bias.reshape(1, f))
    return out[:n]


# speedup vs baseline: 1.2365x; 1.2365x over previous
import functools

import jax
import jax.numpy as jnp
from jax import lax
from jax.experimental import pallas as pl
from jax.experimental.pallas import tpu as pltpu
from jax.experimental.pallas import tpu_sc as plsc

NC = 2
NS = 16
NW = NC * NS
CH = 128
NBUF = 2
TB = 1280


def _mesh():
    return plsc.VectorSubcoreMesh(
        core_axis_name="c", subcore_axis_name="s", num_cores=NC,
        num_subcores=NS)


@functools.lru_cache(maxsize=None)
def _sc_degree(n_pad, nch):
    rows_pt = n_pad // NS

    def body(row_hbm, ones_hbm, zrow_hbm, degp_hbm, ones_v, idx_v, deg_sh,
             sem):
        c = lax.axis_index("c")
        s = lax.axis_index("s")
        wid = s * NC + c
        pltpu.sync_copy(zrow_hbm, deg_sh.at[pl.ds(s * rows_pt, rows_pt)])
        pltpu.sync_copy(ones_hbm, ones_v)
        pltpu.sync_copy(row_hbm.at[wid], idx_v)
        plsc.subcore_barrier()

        def group(g, carry):
            for k in range(8):
                pltpu.async_copy(ones_v, deg_sh.at[idx_v.at[g * 8 + k]], sem,
                                 add=True)
            for k in range(8):
                pltpu.make_async_copy(
                    ones_v, deg_sh.at[idx_v.at[g * 8 + k]], sem).wait()
            return carry

        lax.fori_loop(0, nch // 8, group, 0)
        plsc.subcore_barrier()
        pltpu.sync_copy(deg_sh.at[pl.ds(s * rows_pt, rows_pt)],
                        degp_hbm.at[pl.ds(c * n_pad + s * rows_pt, rows_pt)])

    return pl.kernel(
        body,
        out_type=jax.ShapeDtypeStruct((NC * n_pad,), jnp.float32),
        mesh=_mesh(),
        scratch_types=[
            pltpu.VMEM((CH,), jnp.float32),
            pltpu.VMEM((nch, CH), jnp.int32),
            pltpu.VMEM_SHARED((n_pad,), jnp.float32),
            pltpu.SemaphoreType.DMA,
        ],
    )


@functools.lru_cache(maxsize=None)
def _sc_spmm(n_pad, f, nch):
    rows_pt = n_pad // NS
    zch = rows_pt // 5

    def body(tab_hbm, row_hbm, col_hbm, zblk_hbm, outp_hbm, cidx_v, ridx_v,
             bufs, acc_sh, csem, gsems, rsems, ssems):
        c = lax.axis_index("c")
        s = lax.axis_index("s")
        wid = s * NC + c
        cpre = pltpu.async_copy(col_hbm.at[wid], cidx_v, csem)
        for k in range(NBUF):
            pltpu.async_copy(row_hbm.at[wid, k], ridx_v.at[k], rsems[k])
        for j in range(5):
            pltpu.sync_copy(zblk_hbm,
                            acc_sh.at[pl.ds(s * rows_pt + j * zch, zch)])
        cpre.wait()
        plsc.subcore_barrier()

        for k in range(NBUF):
            pltpu.async_copy(tab_hbm.at[cidx_v.at[k]], bufs[k], gsems[k])

        def pair(j, carry):
            base = j * NBUF
            for k in range(NBUF):
                ch = base + k
                pltpu.make_async_copy(
                    tab_hbm.at[cidx_v.at[ch]], bufs[k], gsems[k]).wait()
                pltpu.make_async_copy(
                    row_hbm.at[wid, ch], ridx_v.at[k], rsems[k]).wait()
                pltpu.sync_copy(bufs[k], acc_sh.at[ridx_v.at[k]], add=True)

                @pl.when(ch + NBUF < nch)
                def _():
                    pltpu.async_copy(tab_hbm.at[cidx_v.at[ch + NBUF]],
                                     bufs[k], gsems[k])
                    pltpu.async_copy(row_hbm.at[wid, ch + NBUF],
                                     ridx_v.at[k], rsems[k])
            return carry

        lax.fori_loop(0, nch // NBUF, pair, 0)
        plsc.subcore_barrier()
        for j in range(5):
            src_off = s * rows_pt + j * zch
            pltpu.sync_copy(
                acc_sh.at[pl.ds(src_off, zch)],
                outp_hbm.at[pl.ds(c * n_pad + src_off, zch)])

    return pl.kernel(
        body,
        out_type=jax.ShapeDtypeStruct((NC * n_pad, f), jnp.float32),
        mesh=_mesh(),
        scratch_types=[
            pltpu.VMEM((nch, CH), jnp.int32),
            pltpu.VMEM((NBUF, CH), jnp.int32),
            [pltpu.VMEM((CH, f), jnp.float32)] * NBUF,
            pltpu.VMEM_SHARED((n_pad, f), jnp.float32),
            pltpu.SemaphoreType.DMA,
            [pltpu.SemaphoreType.DMA] * NBUF,
            [pltpu.SemaphoreType.DMA] * NBUF,
            [pltpu.SemaphoreType.DMA] * NBUF,
        ],
    )


def _dis(degpt_ref):
    d = degpt_ref[:, 0:1] + degpt_ref[:, 1:2]
    return jnp.where(d > 0, lax.rsqrt(d), 0.0)


_HI = jax.lax.Precision.HIGHEST


def _prep_body(x_ref, degpt_ref, u_ref):
    u_ref[...] = x_ref[...] * _dis(degpt_ref)


def _mid_body(vp0_ref, vp1_ref, degpt_ref, t1_ref, w_ref):
    dis = _dis(degpt_ref)
    t1 = -((vp0_ref[...] + vp1_ref[...]) * dis)
    t1_ref[...] = t1
    w_ref[...] = t1 * dis


def _final_body(x_ref, t1_ref, zp0_ref, zp1_ref, degpt_ref, wt_ref, b_ref,
                o_ref):
    dis = _dis(degpt_ref)
    srow = -((zp0_ref[...] + zp1_ref[...]) * dis)
    acc = jnp.dot(x_ref[...], wt_ref[0] - wt_ref[2],
                  preferred_element_type=jnp.float32, precision=_HI)
    acc = acc + jnp.dot(t1_ref[...], wt_ref[1],
                        preferred_element_type=jnp.float32, precision=_HI)
    acc = acc + jnp.dot(srow, 2.0 * wt_ref[2],
                        preferred_element_type=jnp.float32, precision=_HI)
    o_ref[...] = acc + b_ref[...]


def _row_blk(f):
    return pl.BlockSpec((TB, f), lambda i: (i, 0))


def _row_blk_hi(n_pad, f):
    return pl.BlockSpec((TB, f), lambda i: (i + n_pad // TB, 0))


def kernel(x, index, weight, bias):
    n, f = x.shape
    e = index.shape[1]
    blk = NS * 8 * 5 * 2
    n_pad = ((n + blk - 1) // blk) * blk
    nch = -(-e // (NW * CH))
    nch = ((nch + 7) // 8) * 8
    e_pad = NW * nch * CH
    pad_i = jnp.arange(e_pad - e, dtype=jnp.int32)
    row = jnp.concatenate(
        [index[0], n + pad_i % (n_pad - n)]).reshape(NW, nch, CH)
    col = jnp.concatenate([index[1], pad_i % n]).reshape(NW, nch, CH)
    xp = jnp.pad(x, ((0, n_pad - n), (0, 0)))
    grid = (n_pad // TB,)

    ones_row = jnp.ones((CH,), jnp.float32)
    zero_row = jnp.zeros((n_pad // NS,), jnp.float32)
    zero_blk = jnp.zeros((n_pad // NS // 5, f), jnp.float32)

    degp = _sc_degree(n_pad, nch)(row, ones_row, zero_row)
    degpt = jnp.stack([degp[:n_pad], degp[n_pad:]], axis=1)

    wt_spec = pl.BlockSpec((3, f, f), lambda i: (0, 0, 0))

    u = pl.pallas_call(
        _prep_body,
        grid=grid,
        in_specs=[_row_blk(f), _row_blk(2)],
        out_specs=_row_blk(f),
        out_shape=jax.ShapeDtypeStruct((n_pad, f), jnp.float32),
    )(xp, degpt)

    vp = _sc_spmm(n_pad, f, nch)(u, row, col, zero_blk)

    t1, w = pl.pallas_call(
        _mid_body,
        grid=grid,
        in_specs=[_row_blk(f), _row_blk_hi(n_pad, f), _row_blk(2)],
        out_specs=(_row_blk(f), _row_blk(f)),
        out_shape=(jax.ShapeDtypeStruct((n_pad, f), jnp.float32),
                   jax.ShapeDtypeStruct((n_pad, f), jnp.float32)),
    )(vp, vp, degpt)

    zp = _sc_spmm(n_pad, f, nch)(w, row, col, zero_blk)

    out = pl.pallas_call(
        _final_body,
        grid=grid,
        in_specs=[_row_blk(f), _row_blk(f), _row_blk(f),
                  _row_blk_hi(n_pad, f), _row_blk(2), wt_spec,
                  pl.BlockSpec((1, f), lambda i: (0, 0))],
        out_specs=_row_blk(f),
        out_shape=jax.ShapeDtypeStruct((n_pad, f), jnp.float32),
    )(xp, t1, zp, zp, degpt, weight, bias.reshape(1, f))
    return out[:n]


# speedup vs baseline: 1.2541x; 1.0142x over previous
import functools

import jax
import jax.numpy as jnp
from jax import lax
from jax.experimental import pallas as pl
from jax.experimental.pallas import tpu as pltpu
from jax.experimental.pallas import tpu_sc as plsc

NC = 2
NS = 16
NW = NC * NS
CH = 128
NBUF = 2
TB = 1280


def _mesh():
    return plsc.VectorSubcoreMesh(
        core_axis_name="c", subcore_axis_name="s", num_cores=NC,
        num_subcores=NS)


@functools.lru_cache(maxsize=None)
def _sc_degree(n_pad, nch):
    rows_pt = n_pad // NS

    def body(row_hbm, ones_hbm, zrow_hbm, degp0_hbm, degp1_hbm, ones_v,
             idx_v, deg_sh,
             sem):
        c = lax.axis_index("c")
        s = lax.axis_index("s")
        wid = s * NC + c
        pltpu.sync_copy(zrow_hbm, deg_sh.at[pl.ds(s * rows_pt, rows_pt)])
        pltpu.sync_copy(ones_hbm, ones_v)
        pltpu.sync_copy(row_hbm.at[wid], idx_v)
        plsc.subcore_barrier()

        def group(g, carry):
            for k in range(8):
                pltpu.async_copy(ones_v, deg_sh.at[idx_v.at[g * 8 + k]], sem,
                                 add=True)
            for k in range(8):
                pltpu.make_async_copy(
                    ones_v, deg_sh.at[idx_v.at[g * 8 + k]], sem).wait()
            return carry

        lax.fori_loop(0, nch // 8, group, 0)
        plsc.subcore_barrier()

        @pl.when(c == 0)
        def _():
            pltpu.sync_copy(deg_sh.at[pl.ds(s * rows_pt, rows_pt)],
                            degp0_hbm.at[pl.ds(s * rows_pt, rows_pt)])

        @pl.when(c == 1)
        def _():
            pltpu.sync_copy(deg_sh.at[pl.ds(s * rows_pt, rows_pt)],
                            degp1_hbm.at[pl.ds(s * rows_pt, rows_pt)])

    return pl.kernel(
        body,
        out_type=(jax.ShapeDtypeStruct((n_pad,), jnp.float32),
                  jax.ShapeDtypeStruct((n_pad,), jnp.float32)),
        mesh=_mesh(),
        scratch_types=[
            pltpu.VMEM((CH,), jnp.float32),
            pltpu.VMEM((nch, CH), jnp.int32),
            pltpu.VMEM_SHARED((n_pad,), jnp.float32),
            pltpu.SemaphoreType.DMA,
        ],
    )


@functools.lru_cache(maxsize=None)
def _sc_spmm(n_pad, f, nch):
    rows_pt = n_pad // NS
    zch = rows_pt // 5

    def body(tab_hbm, row_hbm, col_hbm, zblk_hbm, outp0_hbm, outp1_hbm,
             cidx_v, ridx_v,
             bufs, acc_sh, csem, gsems, rsems, ssems):
        c = lax.axis_index("c")
        s = lax.axis_index("s")
        wid = s * NC + c
        cpre = pltpu.async_copy(col_hbm.at[wid], cidx_v, csem)
        for k in range(NBUF):
            pltpu.async_copy(row_hbm.at[wid, k], ridx_v.at[k], rsems[k])
        for j in range(5):
            pltpu.sync_copy(zblk_hbm,
                            acc_sh.at[pl.ds(s * rows_pt + j * zch, zch)])
        cpre.wait()
        plsc.subcore_barrier()

        for k in range(NBUF):
            pltpu.async_copy(tab_hbm.at[cidx_v.at[k]], bufs[k], gsems[k])

        def pair(j, carry):
            base = j * NBUF
            for k in range(NBUF):
                ch = base + k
                pltpu.make_async_copy(
                    tab_hbm.at[cidx_v.at[ch]], bufs[k], gsems[k]).wait()
                pltpu.make_async_copy(
                    row_hbm.at[wid, ch], ridx_v.at[k], rsems[k]).wait()
                pltpu.sync_copy(bufs[k], acc_sh.at[ridx_v.at[k]], add=True)

                @pl.when(ch + NBUF < nch)
                def _():
                    pltpu.async_copy(tab_hbm.at[cidx_v.at[ch + NBUF]],
                                     bufs[k], gsems[k])
                    pltpu.async_copy(row_hbm.at[wid, ch + NBUF],
                                     ridx_v.at[k], rsems[k])
            return carry

        lax.fori_loop(0, nch // NBUF, pair, 0)
        plsc.subcore_barrier()

        @pl.when(c == 0)
        def _():
            for j in range(5):
                src_off = s * rows_pt + j * zch
                pltpu.sync_copy(acc_sh.at[pl.ds(src_off, zch)],
                                outp0_hbm.at[pl.ds(src_off, zch)])

        @pl.when(c == 1)
        def _():
            for j in range(5):
                src_off = s * rows_pt + j * zch
                pltpu.sync_copy(acc_sh.at[pl.ds(src_off, zch)],
                                outp1_hbm.at[pl.ds(src_off, zch)])

    return pl.kernel(
        body,
        out_type=(jax.ShapeDtypeStruct((n_pad, f), jnp.float32),
                  jax.ShapeDtypeStruct((n_pad, f), jnp.float32)),
        mesh=_mesh(),
        scratch_types=[
            pltpu.VMEM((nch, CH), jnp.int32),
            pltpu.VMEM((NBUF, CH), jnp.int32),
            [pltpu.VMEM((CH, f), jnp.float32)] * NBUF,
            pltpu.VMEM_SHARED((n_pad, f), jnp.float32),
            pltpu.SemaphoreType.DMA,
            [pltpu.SemaphoreType.DMA] * NBUF,
            [pltpu.SemaphoreType.DMA] * NBUF,
            [pltpu.SemaphoreType.DMA] * NBUF,
        ],
    )


def _dis(degpt_ref):
    d = degpt_ref[:, 0:1] + degpt_ref[:, 1:2]
    return jnp.where(d > 0, lax.rsqrt(d), 0.0)


_HI = jax.lax.Precision.HIGHEST


def _prep_body(x_ref, degpt_ref, u_ref):
    u_ref[...] = x_ref[...] * _dis(degpt_ref)


def _mid_body(vp0_ref, vp1_ref, degpt_ref, t1_ref, w_ref):
    dis = _dis(degpt_ref)
    t1 = -((vp0_ref[...] + vp1_ref[...]) * dis)
    t1_ref[...] = t1
    w_ref[...] = t1 * dis


def _final_body(x_ref, t1_ref, zp0_ref, zp1_ref, degpt_ref, wt_ref, b_ref,
                o_ref):
    dis = _dis(degpt_ref)
    srow = -((zp0_ref[...] + zp1_ref[...]) * dis)
    acc = jnp.dot(x_ref[...], wt_ref[0] - wt_ref[2],
                  preferred_element_type=jnp.float32, precision=_HI)
    acc = acc + jnp.dot(t1_ref[...], wt_ref[1],
                        preferred_element_type=jnp.float32, precision=_HI)
    acc = acc + jnp.dot(srow, 2.0 * wt_ref[2],
                        preferred_element_type=jnp.float32, precision=_HI)
    o_ref[...] = acc + b_ref[...]


def _row_blk(tb, f):
    return pl.BlockSpec((tb, f), lambda i: (i, 0))


def kernel(x, index, weight, bias):
    n, f = x.shape
    e = index.shape[1]
    blk = NS * 8 * 5 * 2
    n_pad = ((n + blk - 1) // blk) * blk
    tb = n // 5
    grid = (n // tb,)
    nch = -(-e // (NW * CH))
    nch = ((nch + 7) // 8) * 8
    e_pad = NW * nch * CH
    pad_i = jnp.arange(e_pad - e, dtype=jnp.int32)
    row = jnp.concatenate(
        [index[0], n + pad_i % (n_pad - n)]).reshape(NW, nch, CH)
    col = jnp.concatenate([index[1], pad_i % n]).reshape(NW, nch, CH)

    ones_row = jnp.ones((CH,), jnp.float32)
    zero_row = jnp.zeros((n_pad // NS,), jnp.float32)
    zero_blk = jnp.zeros((n_pad // NS // 5, f), jnp.float32)

    d0, d1 = _sc_degree(n_pad, nch)(row, ones_row, zero_row)
    degpt = jnp.stack([d0[:n], d1[:n]], axis=1)

    wt_spec = pl.BlockSpec((3, f, f), lambda i: (0, 0, 0))

    u = pl.pallas_call(
        _prep_body,
        grid=grid,
        in_specs=[_row_blk(tb, f), _row_blk(tb, 2)],
        out_specs=_row_blk(tb, f),
        out_shape=jax.ShapeDtypeStruct((n, f), jnp.float32),
    )(x, degpt)

    v0, v1 = _sc_spmm(n_pad, f, nch)(u, row, col, zero_blk)

    t1, w = pl.pallas_call(
        _mid_body,
        grid=grid,
        in_specs=[_row_blk(tb, f), _row_blk(tb, f), _row_blk(tb, 2)],
        out_specs=(_row_blk(tb, f), _row_blk(tb, f)),
        out_shape=(jax.ShapeDtypeStruct((n, f), jnp.float32),
                   jax.ShapeDtypeStruct((n, f), jnp.float32)),
    )(v0, v1, degpt)

    z0, z1 = _sc_spmm(n_pad, f, nch)(w, row, col, zero_blk)

    out = pl.pallas_call(
        _final_body,
        grid=grid,
        in_specs=[_row_blk(tb, f), _row_blk(tb, f), _row_blk(tb, f),
                  _row_blk(tb, f), _row_blk(tb, 2), wt_spec,
                  pl.BlockSpec((1, f), lambda i: (0, 0))],
        out_specs=_row_blk(tb, f),
        out_shape=jax.ShapeDtypeStruct((n, f), jnp.float32),
    )(x, t1, z0, z1, degpt, weight, bias.reshape(1, f))
    return out


# speedup vs baseline: 1.3034x; 1.0394x over previous
import functools

import jax
import jax.numpy as jnp
from jax import lax
from jax.experimental import pallas as pl
from jax.experimental.pallas import tpu as pltpu
from jax.experimental.pallas import tpu_sc as plsc

NC = 2
NS = 16
NW = NC * NS
CH = 128
NBUF = 2
TB = 1280


def _mesh():
    return plsc.VectorSubcoreMesh(
        core_axis_name="c", subcore_axis_name="s", num_cores=NC,
        num_subcores=NS)


@functools.lru_cache(maxsize=None)
def _sc_degree(n_pad, nch):
    rows_pt = n_pad // NS

    def body(row_hbm, ones_hbm, zrow_hbm, degp0_hbm, degp1_hbm, ones_v,
             idx_v, deg_sh,
             sem):
        c = lax.axis_index("c")
        s = lax.axis_index("s")
        wid = s * NC + c
        pltpu.sync_copy(zrow_hbm, deg_sh.at[pl.ds(s * rows_pt, rows_pt)])
        pltpu.sync_copy(ones_hbm, ones_v)
        pltpu.sync_copy(row_hbm.at[wid], idx_v)
        plsc.subcore_barrier()

        def group(g, carry):
            for k in range(8):
                pltpu.async_copy(ones_v, deg_sh.at[idx_v.at[g * 8 + k]], sem,
                                 add=True)
            for k in range(8):
                pltpu.make_async_copy(
                    ones_v, deg_sh.at[idx_v.at[g * 8 + k]], sem).wait()
            return carry

        lax.fori_loop(0, nch // 8, group, 0)
        plsc.subcore_barrier()

        @pl.when(c == 0)
        def _():
            pltpu.sync_copy(deg_sh.at[pl.ds(s * rows_pt, rows_pt)],
                            degp0_hbm.at[pl.ds(s * rows_pt, rows_pt)])

        @pl.when(c == 1)
        def _():
            pltpu.sync_copy(deg_sh.at[pl.ds(s * rows_pt, rows_pt)],
                            degp1_hbm.at[pl.ds(s * rows_pt, rows_pt)])

    return pl.kernel(
        body,
        out_type=(jax.ShapeDtypeStruct((n_pad,), jnp.float32),
                  jax.ShapeDtypeStruct((n_pad,), jnp.float32)),
        mesh=_mesh(),
        scratch_types=[
            pltpu.VMEM((CH,), jnp.float32),
            pltpu.VMEM((nch, CH), jnp.int32),
            pltpu.VMEM_SHARED((n_pad,), jnp.float32),
            pltpu.SemaphoreType.DMA,
        ],
    )


@functools.lru_cache(maxsize=None)
def _sc_spmm(n_pad, f, nch):
    rows_pt = n_pad // NS

    def body(tab_hbm, row_hbm, col_hbm, zblk_hbm, outp0_hbm, outp1_hbm,
             cidx_v, ridx_v,
             bufs, acc_sh, csem, gsems, rsems, ssems):
        c = lax.axis_index("c")
        s = lax.axis_index("s")
        wid = s * NC + c
        cpre = pltpu.async_copy(col_hbm.at[wid], cidx_v, csem)
        for k in range(NBUF):
            pltpu.async_copy(row_hbm.at[wid, k], ridx_v.at[k], rsems[k])
        pltpu.sync_copy(zblk_hbm, acc_sh.at[pl.ds(s * rows_pt, rows_pt)])
        cpre.wait()
        plsc.subcore_barrier()

        for k in range(NBUF):
            pltpu.async_copy(tab_hbm.at[cidx_v.at[k]], bufs[k], gsems[k])

        def pair(j, carry):
            base = j * NBUF
            for k in range(NBUF):
                ch = base + k
                pltpu.make_async_copy(
                    tab_hbm.at[cidx_v.at[ch]], bufs[k], gsems[k]).wait()
                pltpu.make_async_copy(
                    row_hbm.at[wid, ch], ridx_v.at[k], rsems[k]).wait()
                pltpu.sync_copy(bufs[k], acc_sh.at[ridx_v.at[k]], add=True)

                @pl.when(ch + NBUF < nch)
                def _():
                    pltpu.async_copy(tab_hbm.at[cidx_v.at[ch + NBUF]],
                                     bufs[k], gsems[k])
                    pltpu.async_copy(row_hbm.at[wid, ch + NBUF],
                                     ridx_v.at[k], rsems[k])
            return carry

        lax.fori_loop(0, nch // NBUF, pair, 0)
        plsc.subcore_barrier()

        @pl.when(c == 0)
        def _():
            pltpu.sync_copy(acc_sh.at[pl.ds(s * rows_pt, rows_pt)],
                            outp0_hbm.at[pl.ds(s * rows_pt, rows_pt)])

        @pl.when(c == 1)
        def _():
            pltpu.sync_copy(acc_sh.at[pl.ds(s * rows_pt, rows_pt)],
                            outp1_hbm.at[pl.ds(s * rows_pt, rows_pt)])

    return pl.kernel(
        body,
        out_type=(jax.ShapeDtypeStruct((n_pad, f), jnp.float32),
                  jax.ShapeDtypeStruct((n_pad, f), jnp.float32)),
        mesh=_mesh(),
        scratch_types=[
            pltpu.VMEM((nch, CH), jnp.int32),
            pltpu.VMEM((NBUF, CH), jnp.int32),
            [pltpu.VMEM((CH, f), jnp.float32)] * NBUF,
            pltpu.VMEM_SHARED((n_pad, f), jnp.float32),
            pltpu.SemaphoreType.DMA,
            [pltpu.SemaphoreType.DMA] * NBUF,
            [pltpu.SemaphoreType.DMA] * NBUF,
            [pltpu.SemaphoreType.DMA] * NBUF,
        ],
    )


def _dis(degpt_ref):
    d = degpt_ref[:, 0:1] + degpt_ref[:, 1:2]
    return jnp.where(d > 0, lax.rsqrt(d), 0.0)


_HI = jax.lax.Precision.HIGHEST


def _prep_body(x_ref, degpt_ref, u_ref):
    u_ref[...] = x_ref[...] * _dis(degpt_ref)


def _mid_body(vp0_ref, vp1_ref, degpt_ref, t1_ref, w_ref):
    dis = _dis(degpt_ref)
    t1 = -((vp0_ref[...] + vp1_ref[...]) * dis)
    t1_ref[...] = t1
    w_ref[...] = t1 * dis


def _final_body(x_ref, t1_ref, zp0_ref, zp1_ref, degpt_ref, wt_ref, b_ref,
                o_ref):
    dis = _dis(degpt_ref)
    srow = -((zp0_ref[...] + zp1_ref[...]) * dis)
    acc = jnp.dot(x_ref[...], wt_ref[0] - wt_ref[2],
                  preferred_element_type=jnp.float32, precision=_HI)
    acc = acc + jnp.dot(t1_ref[...], wt_ref[1],
                        preferred_element_type=jnp.float32, precision=_HI)
    acc = acc + jnp.dot(srow, 2.0 * wt_ref[2],
                        preferred_element_type=jnp.float32, precision=_HI)
    o_ref[...] = acc + b_ref[...]


def _row_blk(tb, f):
    return pl.BlockSpec((tb, f), lambda i: (i, 0))


def kernel(x, index, weight, bias):
    n, f = x.shape
    e = index.shape[1]
    blk = NS * 8 * 5 * 2
    n_pad = ((n + blk - 1) // blk) * blk
    tb = n // 5
    grid = (n // tb,)
    nch = -(-e // (NW * CH))
    nch = ((nch + 7) // 8) * 8
    e_pad = NW * nch * CH
    pad_i = jnp.arange(e_pad - e, dtype=jnp.int32)
    row = jnp.concatenate(
        [index[0], n + pad_i % (n_pad - n)]).reshape(NW, nch, CH)
    col = jnp.concatenate([index[1], pad_i % n]).reshape(NW, nch, CH)

    ones_row = jnp.ones((CH,), jnp.float32)
    zero_row = jnp.zeros((n_pad // NS,), jnp.float32)
    zero_blk = jnp.zeros((n_pad // NS, f), jnp.float32)

    d0, d1 = _sc_degree(n_pad, nch)(row, ones_row, zero_row)
    degpt = jnp.stack([d0[:n], d1[:n]], axis=1)

    wt_spec = pl.BlockSpec((3, f, f), lambda i: (0, 0, 0))

    u = pl.pallas_call(
        _prep_body,
        grid=grid,
        in_specs=[_row_blk(tb, f), _row_blk(tb, 2)],
        out_specs=_row_blk(tb, f),
        out_shape=jax.ShapeDtypeStruct((n, f), jnp.float32),
    )(x, degpt)

    v0, v1 = _sc_spmm(n_pad, f, nch)(u, row, col, zero_blk)

    t1, w = pl.pallas_call(
        _mid_body,
        grid=grid,
        in_specs=[_row_blk(tb, f), _row_blk(tb, f), _row_blk(tb, 2)],
        out_specs=(_row_blk(tb, f), _row_blk(tb, f)),
        out_shape=(jax.ShapeDtypeStruct((n, f), jnp.float32),
                   jax.ShapeDtypeStruct((n, f), jnp.float32)),
    )(v0, v1, degpt)

    z0, z1 = _sc_spmm(n_pad, f, nch)(w, row, col, zero_blk)

    out = pl.pallas_call(
        _final_body,
        grid=grid,
        in_specs=[_row_blk(tb, f), _row_blk(tb, f), _row_blk(tb, f),
                  _row_blk(tb, f), _row_blk(tb, 2), wt_spec,
                  pl.BlockSpec((1, f), lambda i: (0, 0))],
        out_specs=_row_blk(tb, f),
        out_shape=jax.ShapeDtypeStruct((n, f), jnp.float32),
    )(x, t1, z0, z1, degpt, weight, bias.reshape(1, f))
    return out


# speedup vs baseline: 1.3036x; 1.0001x over previous
import functools

import jax
import jax.numpy as jnp
from jax import lax
from jax.experimental import pallas as pl
from jax.experimental.pallas import tpu as pltpu
from jax.experimental.pallas import tpu_sc as plsc

NC = 2
NS = 16
NW = NC * NS
CH = 128
NBUF = 2


def _mesh():
    return plsc.VectorSubcoreMesh(
        core_axis_name="c", subcore_axis_name="s", num_cores=NC,
        num_subcores=NS)


@functools.lru_cache(maxsize=None)
def _sc_degree(n_pad, nch):
    rows_pt = n_pad // NS

    def body(row_hbm, ones_hbm, zrow_hbm, degp0_hbm, degp1_hbm, ones_v,
             idx_v, deg_sh, sem):
        c = lax.axis_index("c")
        s = lax.axis_index("s")
        wid = s * NC + c
        pltpu.sync_copy(zrow_hbm, deg_sh.at[pl.ds(s * rows_pt, rows_pt)])
        pltpu.sync_copy(ones_hbm, ones_v)
        pltpu.sync_copy(row_hbm.at[wid], idx_v)
        plsc.subcore_barrier()

        def group(g, carry):
            for k in range(8):
                pltpu.async_copy(ones_v, deg_sh.at[idx_v.at[g * 8 + k]], sem,
                                 add=True)
            for k in range(8):
                pltpu.make_async_copy(
                    ones_v, deg_sh.at[idx_v.at[g * 8 + k]], sem).wait()
            return carry

        lax.fori_loop(0, nch // 8, group, 0)
        plsc.subcore_barrier()

        @pl.when(c == 0)
        def _():
            pltpu.sync_copy(deg_sh.at[pl.ds(s * rows_pt, rows_pt)],
                            degp0_hbm.at[pl.ds(s * rows_pt, rows_pt)])

        @pl.when(c == 1)
        def _():
            pltpu.sync_copy(deg_sh.at[pl.ds(s * rows_pt, rows_pt)],
                            degp1_hbm.at[pl.ds(s * rows_pt, rows_pt)])

    return pl.kernel(
        body,
        out_type=(jax.ShapeDtypeStruct((n_pad,), jnp.float32),
                  jax.ShapeDtypeStruct((n_pad,), jnp.float32)),
        mesh=_mesh(),
        scratch_types=[
            pltpu.VMEM((CH,), jnp.float32),
            pltpu.VMEM((nch, CH), jnp.int32),
            pltpu.VMEM_SHARED((n_pad,), jnp.float32),
            pltpu.SemaphoreType.DMA,
        ],
    )


@functools.lru_cache(maxsize=None)
def _sc_spmm(n_pad, f, nch):
    rows_pt = n_pad // NS

    def body(tab_hbm, row_hbm, col_hbm, zblk_hbm, outp0_hbm, outp1_hbm,
             cidx_v, ridx_v, bufs, acc_sh, csem, gsems, rsems):
        c = lax.axis_index("c")
        s = lax.axis_index("s")
        wid = s * NC + c
        cpre = pltpu.async_copy(col_hbm.at[wid], cidx_v, csem)
        for k in range(NBUF):
            pltpu.async_copy(row_hbm.at[wid, k], ridx_v.at[k], rsems[k])
        pltpu.sync_copy(zblk_hbm, acc_sh.at[pl.ds(s * rows_pt, rows_pt)])
        cpre.wait()
        plsc.subcore_barrier()

        for k in range(NBUF):
            pltpu.async_copy(tab_hbm.at[cidx_v.at[k]], bufs[k], gsems[k])

        def pair(j, carry):
            base = j * NBUF
            for k in range(NBUF):
                ch = base + k
                pltpu.make_async_copy(
                    tab_hbm.at[cidx_v.at[ch]], bufs[k], gsems[k]).wait()
                pltpu.make_async_copy(
                    row_hbm.at[wid, ch], ridx_v.at[k], rsems[k]).wait()
                pltpu.sync_copy(bufs[k], acc_sh.at[ridx_v.at[k]], add=True)

                @pl.when(ch + NBUF < nch)
                def _():
                    pltpu.async_copy(tab_hbm.at[cidx_v.at[ch + NBUF]],
                                     bufs[k], gsems[k])
                    pltpu.async_copy(row_hbm.at[wid, ch + NBUF],
                                     ridx_v.at[k], rsems[k])
            return carry

        lax.fori_loop(0, nch // NBUF, pair, 0)
        plsc.subcore_barrier()

        @pl.when(c == 0)
        def _():
            pltpu.sync_copy(acc_sh.at[pl.ds(s * rows_pt, rows_pt)],
                            outp0_hbm.at[pl.ds(s * rows_pt, rows_pt)])

        @pl.when(c == 1)
        def _():
            pltpu.sync_copy(acc_sh.at[pl.ds(s * rows_pt, rows_pt)],
                            outp1_hbm.at[pl.ds(s * rows_pt, rows_pt)])

    return pl.kernel(
        body,
        out_type=(jax.ShapeDtypeStruct((n_pad, f), jnp.float32),
                  jax.ShapeDtypeStruct((n_pad, f), jnp.float32)),
        mesh=_mesh(),
        scratch_types=[
            pltpu.VMEM((nch, CH), jnp.int32),
            pltpu.VMEM((NBUF, CH), jnp.int32),
            [pltpu.VMEM((CH, f), jnp.float32)] * NBUF,
            pltpu.VMEM_SHARED((n_pad, f), jnp.float32),
            pltpu.SemaphoreType.DMA,
            [pltpu.SemaphoreType.DMA] * NBUF,
            [pltpu.SemaphoreType.DMA] * NBUF,
        ],
    )


def _dis(degpt_ref):
    d = degpt_ref[:, 0:1] + degpt_ref[:, 1:2]
    return jnp.where(d > 0, lax.rsqrt(d), 0.0)


_HI = jax.lax.Precision.HIGHEST


def _prep_body(x_ref, degpt_ref, u_ref):
    u_ref[...] = x_ref[...] * _dis(degpt_ref)


def _mid_body(vp0_ref, vp1_ref, degpt_ref, t1_ref, w_ref):
    dis = _dis(degpt_ref)
    t1 = -((vp0_ref[...] + vp1_ref[...]) * dis)
    t1_ref[...] = t1
    w_ref[...] = t1 * dis


def _final_body(x_ref, t1_ref, zp0_ref, zp1_ref, degpt_ref, wt_ref, b_ref,
                o_ref):
    dis = _dis(degpt_ref)
    srow = -((zp0_ref[...] + zp1_ref[...]) * dis)
    acc = jnp.dot(x_ref[...], wt_ref[0] - wt_ref[2],
                  preferred_element_type=jnp.float32, precision=_HI)
    acc = acc + jnp.dot(t1_ref[...], wt_ref[1],
                        preferred_element_type=jnp.float32, precision=_HI)
    acc = acc + jnp.dot(srow, 2.0 * wt_ref[2],
                        preferred_element_type=jnp.float32, precision=_HI)
    o_ref[...] = acc + b_ref[...]


def _row_blk(tb, f):
    return pl.BlockSpec((tb, f), lambda i: (i, 0))


def kernel(x, index, weight, bias):
    n, f = x.shape
    e = index.shape[1]
    blk = NS * 8 * 5 * 2
    n_pad = ((n + blk - 1) // blk) * blk
    tb = n // 5
    grid = (n // tb,)
    nch = -(-e // (NW * CH))
    nch = ((nch + 7) // 8) * 8
    e_pad = NW * nch * CH
    pad_i = jnp.arange(e_pad - e, dtype=jnp.int32)
    row = jnp.concatenate(
        [index[0], n + pad_i % (n_pad - n)]).reshape(NW, nch, CH)
    col = jnp.concatenate([index[1], pad_i % n]).reshape(NW, nch, CH)

    ones_row = jnp.ones((CH,), jnp.float32)
    zero_row = jnp.zeros((n_pad // NS,), jnp.float32)
    zero_blk = jnp.zeros((n_pad // NS, f), jnp.float32)

    d0, d1 = _sc_degree(n_pad, nch)(row, ones_row, zero_row)
    degpt = jnp.stack([d0[:n], d1[:n]], axis=1)

    wt_spec = pl.BlockSpec((3, f, f), lambda i: (0, 0, 0))

    u = pl.pallas_call(
        _prep_body,
        grid=grid,
        in_specs=[_row_blk(tb, f), _row_blk(tb, 2)],
        out_specs=_row_blk(tb, f),
        out_shape=jax.ShapeDtypeStruct((n, f), jnp.float32),
    )(x, degpt)

    v0, v1 = _sc_spmm(n_pad, f, nch)(u, row, col, zero_blk)

    t1, w = pl.pallas_call(
        _mid_body,
        grid=grid,
        in_specs=[_row_blk(tb, f), _row_blk(tb, f), _row_blk(tb, 2)],
        out_specs=(_row_blk(tb, f), _row_blk(tb, f)),
        out_shape=(jax.ShapeDtypeStruct((n, f), jnp.float32),
                   jax.ShapeDtypeStruct((n, f), jnp.float32)),
    )(v0, v1, degpt)

    z0, z1 = _sc_spmm(n_pad, f, nch)(w, row, col, zero_blk)

    out = pl.pallas_call(
        _final_body,
        grid=grid,
        in_specs=[_row_blk(tb, f), _row_blk(tb, f), _row_blk(tb, f),
                  _row_blk(tb, f), _row_blk(tb, 2), wt_spec,
                  pl.BlockSpec((1, f), lambda i: (0, 0))],
        out_specs=_row_blk(tb, f),
        out_shape=jax.ShapeDtypeStruct((n, f), jnp.float32),
    )(x, t1, z0, z1, degpt, weight, bias.reshape(1, f))
    return out
